# baseline (speedup 1.0000x reference)
import functools

import jax
import jax.numpy as jnp
from jax import lax
from jax.experimental import pallas as pl
from jax.experimental.pallas import tpu as pltpu

LEAKY = 0.1


def _shift_leaky(y, sh):
    y = y + sh
    return jnp.where(y > 0, y, LEAKY * y)


def _conv1x1(x, w_ref, cin, c, sc_ref, sh_ref):
    w = (w_ref[:cin, :c] * sc_ref[...]).astype(jnp.bfloat16)
    y = lax.dot_general(x.astype(jnp.bfloat16), w,
                        (((0,), (0,)), ((), ())),
                        preferred_element_type=jnp.float32)
    return _shift_leaky(y, sh_ref[...])


def _merge3x3(s, col, m_ref, cp, msc_ref, msh_ref, H, W, C):
    sb = s.astype(col.dtype)
    col[0:1, :, :] = jnp.zeros_like(col[0:1, :, :])
    col[H + 1:H + 2, :, :] = jnp.zeros_like(col[H + 1:H + 2, :, :])
    col[1:H + 1, :, C:2 * C] = sb
    col[1:H + 1, 0:1, 0:C] = jnp.zeros_like(col[1:H + 1, 0:1, 0:C])
    col[1:H + 1, 1:W, 0:C] = sb[:, 0:W - 1, :]
    col[1:H + 1, W - 1:W, 2 * C:3 * C] = jnp.zeros_like(
        col[1:H + 1, W - 1:W, 2 * C:3 * C])
    col[1:H + 1, 0:W - 1, 2 * C:3 * C] = sb[:, 1:W, :]
    acc = jnp.zeros((C, H * W), jnp.float32)
    for dy in range(3):
        mt = (jnp.concatenate(
            [m_ref[(3 * dy + dx) * cp:(3 * dy + dx) * cp + C, :C]
             for dx in range(3)], axis=0) * msc_ref[...]).astype(col.dtype)
        win = col[dy:dy + H].reshape(H * W, 3 * C)
        acc = acc + lax.dot_general(mt, win, (((0,), (1,)), ((), ())),
                                    preferred_element_type=jnp.float32)
    return _shift_leaky(acc, msh_ref[...].T)


def _up2x(o, H, W, C):
    return jnp.repeat(jnp.repeat(o, 2, axis=0), 2, axis=1)


def _fpn_body(x3_ref, x2_ref, x1_ref, w3_ref, w2_ref, w1_ref, m2_ref, m1_ref,
              sc3, sh3, sc2, sh2, sc1, sh1, msc2, msh2, msc1, msh1,
              oc1_ref, oc2_ref, oc3_ref, spad1, spad2,
              *, C, CP, H3, W3, H2, W2, H1, W1, C3, C2, C1):
    y3 = _conv1x1(x3_ref[0], w3_ref, C3, C, sc3, sh3)
    oc3_ref[0] = y3.T

    s2 = _conv1x1(x2_ref[0], w2_ref, C2, C, sc2, sh2)
    s2 = s2.reshape(H2, W2, C) + _up2x(y3.reshape(H3, W3, C), H2, W2, C)
    y2 = _merge3x3(s2, spad2, m2_ref, CP, msc2, msh2, H2, W2, C)
    oc2_ref[0] = y2

    x1 = x1_ref[0].reshape(C1, H1 * W1)
    s1 = _conv1x1(x1, w1_ref, C1, C, sc1, sh1)
    s1 = s1.reshape(H1, W1, C) + _up2x(y2.T.reshape(H2, W2, C), H1, W1, C)
    y1 = _merge3x3(s1, spad1, m1_ref, CP, msc1, msh1, H1, W1, C)
    oc1_ref[0] = y1.reshape(C, H1, W1)


def kernel(x1, x2, x3,
           output1_w, output1_scale, output1_shift,
           output1_w_torch, output1_scale_raw, output1_shift_raw,
           output2_w, output2_scale, output2_shift,
           output2_w_torch, output2_scale_raw, output2_shift_raw,
           output3_w, output3_scale, output3_shift,
           output3_w_torch, output3_scale_raw, output3_shift_raw,
           merge1_w, merge1_scale, merge1_shift,
           merge1_w_torch, merge1_scale_raw, merge1_shift_raw,
           merge2_w, merge2_scale, merge2_shift,
           merge2_w_torch, merge2_scale_raw, merge2_shift_raw):
    N, C1, H1, W1 = x1.shape
    _, C2, H2, W2 = x2.shape
    _, C3, H3, W3 = x3.shape
    C = merge1_w_torch.shape[0]
    CP = merge1_w.shape[0] // 9

    body = functools.partial(
        _fpn_body, C=C, CP=CP, H3=H3, W3=W3, H2=H2, W2=W2, H1=H1, W1=W1,
        C3=C3, C2=C2, C1=C1)

    row = lambda v: v.reshape(1, C)
    full = lambda shp: pl.BlockSpec(shp, lambda n: tuple(0 for _ in shp))

    oc1, oc2, oc3 = pl.pallas_call(
        body,
        grid=(N,),
        in_specs=[
            pl.BlockSpec((1, C3, H3 * W3), lambda n: (n, 0, 0)),
            pl.BlockSpec((1, C2, H2 * W2), lambda n: (n, 0, 0)),
            pl.BlockSpec((1, C1, H1, W1), lambda n: (n, 0, 0, 0)),
            full(output3_w.shape),
            full(output2_w.shape),
            full(output1_w.shape),
            full(merge2_w.shape),
            full(merge1_w.shape),
        ] + [full((1, C))] * 10,
        out_specs=(
            pl.BlockSpec((1, C, H1, W1), lambda n: (n, 0, 0, 0)),
            pl.BlockSpec((1, C, H2 * W2), lambda n: (n, 0, 0)),
            pl.BlockSpec((1, C, H3 * W3), lambda n: (n, 0, 0)),
        ),
        out_shape=(
            jax.ShapeDtypeStruct((N, C, H1, W1), jnp.float32),
            jax.ShapeDtypeStruct((N, C, H2 * W2), jnp.float32),
            jax.ShapeDtypeStruct((N, C, H3 * W3), jnp.float32),
        ),
        scratch_shapes=[
            pltpu.VMEM((H1 + 2, W1, 3 * C), jnp.bfloat16),
            pltpu.VMEM((H2 + 2, W2, 3 * C), jnp.bfloat16),
        ],
        compiler_params=pltpu.CompilerParams(
            dimension_semantics=("parallel",)),
    )(x3.reshape(N, C3, H3 * W3),
      x2.reshape(N, C2, H2 * W2),
      x1,
      output3_w, output2_w, output1_w, merge2_w, merge1_w,
      row(output3_scale_raw), row(output3_shift_raw),
      row(output2_scale_raw), row(output2_shift_raw),
      row(output1_scale_raw), row(output1_shift_raw),
      row(merge2_scale_raw), row(merge2_shift_raw),
      row(merge1_scale_raw), row(merge1_shift_raw))

    return [oc1,
            oc2.reshape(N, C, H2, W2),
            oc3.reshape(N, C, H3, W3)]


# speedup vs baseline: 1.0123x; 1.0123x over previous
import functools

import jax
import jax.numpy as jnp
from jax import lax
from jax.experimental import pallas as pl
from jax.experimental.pallas import tpu as pltpu

LEAKY = 0.1


def _shift_leaky(y, sh):
    y = y + sh
    return jnp.where(y > 0, y, LEAKY * y)


def _conv1x1(x, w_ref, cin, c, sc_ref, sh_ref):
    w = (w_ref[:cin, :c] * sc_ref[...]).astype(jnp.bfloat16)
    y = lax.dot_general(x.astype(jnp.bfloat16), w,
                        (((0,), (0,)), ((), ())),
                        preferred_element_type=jnp.float32)
    return _shift_leaky(y, sh_ref[...])


def _merge3x3(s, col, m_ref, cp, msc_ref, msh_ref, H, W, C):
    sb = s.astype(col.dtype)
    col[0:1, :, :] = jnp.zeros_like(col[0:1, :, :])
    col[H + 1:H + 2, :, :] = jnp.zeros_like(col[H + 1:H + 2, :, :])
    col[1:H + 1, :, C:2 * C] = sb
    col[1:H + 1, 0:1, 0:C] = jnp.zeros_like(col[1:H + 1, 0:1, 0:C])
    col[1:H + 1, 1:W, 0:C] = sb[:, 0:W - 1, :]
    col[1:H + 1, W - 1:W, 2 * C:3 * C] = jnp.zeros_like(
        col[1:H + 1, W - 1:W, 2 * C:3 * C])
    col[1:H + 1, 0:W - 1, 2 * C:3 * C] = sb[:, 1:W, :]
    acc = jnp.zeros((C, H * W), jnp.float32)
    for dy in range(3):
        mt = (jnp.concatenate(
            [m_ref[(3 * dy + dx) * cp:(3 * dy + dx) * cp + C, :C]
             for dx in range(3)], axis=0) * msc_ref[...]).astype(col.dtype)
        win = col[dy:dy + H].reshape(H * W, 3 * C)
        acc = acc + lax.dot_general(mt, win, (((0,), (1,)), ((), ())),
                                    preferred_element_type=jnp.float32)
    return _shift_leaky(acc, msh_ref[...].T)


def _up2x(o, H, W, C):
    return jnp.repeat(jnp.repeat(o, 2, axis=1), 2, axis=0)


def _fpn_body(x3_ref, x2_ref, x1_ref, w3_ref, w2_ref, w1_ref, m2_ref, m1_ref,
              sc3, sh3, sc2, sh2, sc1, sh1, msc2, msh2, msc1, msh1,
              oc1_ref, oc2_ref, oc3_ref, spad1, spad2,
              *, C, CP, H3, W3, H2, W2, H1, W1, C3, C2, C1):
    y3 = _conv1x1(x3_ref[0], w3_ref, C3, C, sc3, sh3)
    oc3_ref[0] = y3.T

    s2 = _conv1x1(x2_ref[0], w2_ref, C2, C, sc2, sh2)
    s2 = s2.reshape(H2, W2, C) + _up2x(y3.reshape(H3, W3, C), H2, W2, C)
    y2 = _merge3x3(s2, spad2, m2_ref, CP, msc2, msh2, H2, W2, C)
    oc2_ref[0] = y2

    x1 = x1_ref[0].reshape(C1, H1 * W1)
    s1 = _conv1x1(x1, w1_ref, C1, C, sc1, sh1)
    s1 = s1.reshape(H1, W1, C) + _up2x(y2.T.reshape(H2, W2, C), H1, W1, C)
    y1 = _merge3x3(s1, spad1, m1_ref, CP, msc1, msh1, H1, W1, C)
    oc1_ref[0] = y1.reshape(C, H1, W1)


def kernel(x1, x2, x3,
           output1_w, output1_scale, output1_shift,
           output1_w_torch, output1_scale_raw, output1_shift_raw,
           output2_w, output2_scale, output2_shift,
           output2_w_torch, output2_scale_raw, output2_shift_raw,
           output3_w, output3_scale, output3_shift,
           output3_w_torch, output3_scale_raw, output3_shift_raw,
           merge1_w, merge1_scale, merge1_shift,
           merge1_w_torch, merge1_scale_raw, merge1_shift_raw,
           merge2_w, merge2_scale, merge2_shift,
           merge2_w_torch, merge2_scale_raw, merge2_shift_raw):
    N, C1, H1, W1 = x1.shape
    _, C2, H2, W2 = x2.shape
    _, C3, H3, W3 = x3.shape
    C = merge1_w_torch.shape[0]
    CP = merge1_w.shape[0] // 9

    body = functools.partial(
        _fpn_body, C=C, CP=CP, H3=H3, W3=W3, H2=H2, W2=W2, H1=H1, W1=W1,
        C3=C3, C2=C2, C1=C1)

    row = lambda v: v.reshape(1, C)
    full = lambda shp: pl.BlockSpec(shp, lambda n: tuple(0 for _ in shp))

    oc1, oc2, oc3 = pl.pallas_call(
        body,
        grid=(N,),
        in_specs=[
            pl.BlockSpec((1, C3, H3 * W3), lambda n: (n, 0, 0)),
            pl.BlockSpec((1, C2, H2 * W2), lambda n: (n, 0, 0)),
            pl.BlockSpec((1, C1, H1, W1), lambda n: (n, 0, 0, 0)),
            full(output3_w.shape),
            full(output2_w.shape),
            full(output1_w.shape),
            full(merge2_w.shape),
            full(merge1_w.shape),
        ] + [full((1, C))] * 10,
        out_specs=(
            pl.BlockSpec((1, C, H1, W1), lambda n: (n, 0, 0, 0)),
            pl.BlockSpec((1, C, H2 * W2), lambda n: (n, 0, 0)),
            pl.BlockSpec((1, C, H3 * W3), lambda n: (n, 0, 0)),
        ),
        out_shape=(
            jax.ShapeDtypeStruct((N, C, H1, W1), jnp.float32),
            jax.ShapeDtypeStruct((N, C, H2 * W2), jnp.float32),
            jax.ShapeDtypeStruct((N, C, H3 * W3), jnp.float32),
        ),
        scratch_shapes=[
            pltpu.VMEM((H1 + 2, W1, 3 * C), jnp.bfloat16),
            pltpu.VMEM((H2 + 2, W2, 3 * C), jnp.bfloat16),
        ],
        compiler_params=pltpu.CompilerParams(
            dimension_semantics=("parallel",)),
    )(x3.reshape(N, C3, H3 * W3),
      x2.reshape(N, C2, H2 * W2),
      x1,
      output3_w, output2_w, output1_w, merge2_w, merge1_w,
      row(output3_scale_raw), row(output3_shift_raw),
      row(output2_scale_raw), row(output2_shift_raw),
      row(output1_scale_raw), row(output1_shift_raw),
      row(merge2_scale_raw), row(merge2_shift_raw),
      row(merge1_scale_raw), row(merge1_shift_raw))

    return [oc1,
            oc2.reshape(N, C, H2, W2),
            oc3.reshape(N, C, H3, W3)]
